# baseline (speedup 1.0000x reference)
import functools

import numpy as np
import jax
import jax.numpy as jnp
from jax.experimental import pallas as pl
from jax.experimental.pallas import tpu as pltpu


def _bilinear_matrix(n_in: int, n_out: int) -> np.ndarray:
    M = np.zeros((n_out, n_in), dtype=np.float32)
    if n_out == 1 or n_in == 1:
        M[:, 0] = 1.0
        return M
    scale = (n_in - 1) / (n_out - 1)
    rows = np.arange(n_out)
    src = rows * scale
    i0 = np.minimum(np.floor(src).astype(np.int64), n_in - 1)
    i1 = np.minimum(i0 + 1, n_in - 1)
    f = src - i0
    M[rows, i0] += (1.0 - f).astype(np.float32)
    M[rows, i1] += f.astype(np.float32)
    return M


def _wsum(x_view, w_ref, w_off, C):
    terms = [x_view[c] * w_ref[w_off + c] for c in range(C)]
    while len(terms) > 1:
        nxt = [a + b for a, b in zip(terms[0::2], terms[1::2])]
        if len(terms) % 2:
            nxt.append(terms[-1])
        terms = nxt
    return terms[0]


def _head_kernel(w_ref, b_ref,
                 x4h, x3h, x2h, x1h, x0h, uh_ref, uwt_ref,
                 o0h, o1h, o2h, o3h, o4h,
                 xb4, xb3, xb2, xb1, xb0,
                 ob0, ob1, ob2, ob3, ob4,
                 in_sems, out_sems,
                 *, N, chans, h_sizes):
    obs = [ob0, ob1, ob2, ob3, ob4]
    ohs = [o0h, o1h, o2h, o3h, o4h]
    w_offs = [int(sum(chans[:l])) for l in range(5)]
    u_offs = [int(sum(h_sizes[1:l])) for l in range(1, 5)]
    C3 = 2
    C2 = 4

    def x4_copy(n, slot):
        return pltpu.make_async_copy(x4h.at[n], xb4.at[slot],
                                     in_sems.at[0, slot])

    def x3_copy(c, slot):
        return pltpu.make_async_copy(x3h.at[pl.ds(c * C3, C3)], xb3.at[slot],
                                     in_sems.at[1, slot])

    def x2_copy(c, slot):
        return pltpu.make_async_copy(x2h.at[pl.ds(c * C2, C2)], xb2.at[slot],
                                     in_sems.at[2, slot])

    def x1_copy():
        return pltpu.make_async_copy(x1h, xb1, in_sems.at[3, 0])

    def x0_copy():
        return pltpu.make_async_copy(x0h, xb0, in_sems.at[4, 0])

    x4_copy(0, 0).start()
    x3_copy(0, 0).start()
    x2_copy(0, 0).start()
    x1_copy().start()
    x0_copy().start()

    def body(n, _):
        s2 = jax.lax.rem(n, 2)
        c3 = jax.lax.div(n, C3)
        c2 = jax.lax.div(n, C2)
        s3 = jax.lax.rem(c3, 2)
        s2c = jax.lax.rem(c2, 2)

        @pl.when(n + 1 < N)
        def _pf4():
            x4_copy(n + 1, 1 - s2).start()

        @pl.when(jnp.logical_and(jax.lax.rem(n, C3) == 0, (c3 + 1) * C3 < N))
        def _pf3():
            x3_copy(c3 + 1, 1 - s3).start()

        @pl.when(jnp.logical_and(jax.lax.rem(n, C2) == 0, (c2 + 1) * C2 < N))
        def _pf2():
            x2_copy(c2 + 1, 1 - s2c).start()

        x4_copy(n, s2).wait()
        y0 = _wsum(xb4.at[s2], w_ref, w_offs[0], chans[0])
        obs[0][n, 0] = jax.nn.sigmoid(y0 + b_ref[0])

        def up_level(lvl, x_view):
            C = chans[lvl]
            H = h_sizes[lvl]
            off = u_offs[lvl - 1]
            y = _wsum(x_view, w_ref, w_offs[lvl], C)
            t = jnp.dot(uh_ref[:, off:off + H], y,
                        preferred_element_type=jnp.float32)
            up = jnp.dot(t, uwt_ref[off:off + H, :],
                         preferred_element_type=jnp.float32)
            obs[lvl][n, 0] = jax.nn.sigmoid(up + b_ref[lvl])

        @pl.when(jax.lax.rem(n, C3) == 0)
        def _w3():
            x3_copy(0, s3).wait()
        up_level(1, xb3.at[s3, jax.lax.rem(n, C3)])

        @pl.when(jax.lax.rem(n, C2) == 0)
        def _w2():
            x2_copy(0, s2c).wait()
        up_level(2, xb2.at[s2c, jax.lax.rem(n, C2)])

        @pl.when(n == 0)
        def _w10():
            x1_copy().wait()
            x0_copy().wait()
        up_level(3, xb1.at[n])
        up_level(4, xb0.at[n])
        return 0

    jax.lax.fori_loop(0, N, body, 0)

    for lvl in range(5):
        pltpu.make_async_copy(obs[lvl], ohs[lvl], out_sems.at[lvl]).start()
    for lvl in range(5):
        pltpu.make_async_copy(obs[lvl], ohs[lvl], out_sems.at[lvl]).wait()


def kernel(x0, x1, x2, x3, x4, w0, w1, w2, w3, w4, b0, b1, b2, b3, b4):
    N = x0.shape[0]
    assert N % 4 == 0
    Ho, Wo = x4.shape[2], x4.shape[3]
    xs = [x4, x3, x2, x1, x0]
    chans = tuple(x.shape[1] for x in xs)
    h_sizes = tuple(x.shape[2] for x in xs)

    uh_all = jnp.asarray(np.concatenate(
        [_bilinear_matrix(h, Ho) for h in h_sizes[1:]], axis=1))
    uwt_all = jnp.asarray(np.concatenate(
        [_bilinear_matrix(h, Wo).T for h in h_sizes[1:]], axis=0))

    w_all = jnp.concatenate([w0, w1, w2, w3, w4])
    b_all = jnp.concatenate([b0, b1, b2, b3, b4])

    smem = pl.BlockSpec(memory_space=pltpu.MemorySpace.SMEM)
    anys = pl.BlockSpec(memory_space=pltpu.MemorySpace.HBM)
    vmem = pl.BlockSpec(memory_space=pltpu.MemorySpace.VMEM)

    out_shape = jax.ShapeDtypeStruct((N, 1, Ho, Wo), jnp.float32)
    f32 = jnp.float32

    outs = pl.pallas_call(
        functools.partial(_head_kernel, N=N, chans=chans, h_sizes=h_sizes),
        out_shape=[out_shape] * 5,
        in_specs=[smem, smem] + [anys] * 5 + [vmem, vmem],
        out_specs=[anys] * 5,
        scratch_shapes=(
            [pltpu.VMEM((2,) + x4.shape[1:], f32),
             pltpu.VMEM((2, 2) + x3.shape[1:], f32),
             pltpu.VMEM((2, 4) + x2.shape[1:], f32),
             pltpu.VMEM(x1.shape, f32),
             pltpu.VMEM(x0.shape, f32)]
            + [pltpu.VMEM((N, 1, Ho, Wo), f32) for _ in range(5)]
            + [pltpu.SemaphoreType.DMA((5, 2)), pltpu.SemaphoreType.DMA((5,))]
        ),
        compiler_params=pltpu.CompilerParams(
            vmem_limit_bytes=52 * 1024 * 1024),
    )(w_all, b_all, x4, x3, x2, x1, x0, uh_all, uwt_all)
    return list(outs)


# speedup vs baseline: 1.0700x; 1.0700x over previous
import functools

import numpy as np
import jax
import jax.numpy as jnp
from jax.experimental import pallas as pl
from jax.experimental.pallas import tpu as pltpu


def _bilinear_matrix(n_in: int, n_out: int) -> np.ndarray:
    M = np.zeros((n_out, n_in), dtype=np.float32)
    if n_out == 1 or n_in == 1:
        M[:, 0] = 1.0
        return M
    scale = (n_in - 1) / (n_out - 1)
    rows = np.arange(n_out)
    src = rows * scale
    i0 = np.minimum(np.floor(src).astype(np.int64), n_in - 1)
    i1 = np.minimum(i0 + 1, n_in - 1)
    f = src - i0
    M[rows, i0] += (1.0 - f).astype(np.float32)
    M[rows, i1] += f.astype(np.float32)
    return M


def _wsum(x_view, w_ref, w_off, C):
    terms = [x_view[c] * w_ref[w_off + c] for c in range(C)]
    while len(terms) > 1:
        nxt = [a + b for a, b in zip(terms[0::2], terms[1::2])]
        if len(terms) % 2:
            nxt.append(terms[-1])
        terms = nxt
    return terms[0]


def _head_kernel(w_ref, b_ref,
                 x4h, x3h, x2h, x1h, x0h, uh_ref, uwt_ref,
                 o0h, o1h, o2h, o3h, o4h,
                 xb4, xb3, xb2, xb1, xb0,
                 ob0, ob1, ob2, ob3, ob4,
                 in_sems, out_sems,
                 *, N, chans, h_sizes):
    xhs = [x4h, x3h, x2h, x1h, x0h]
    xbs = [xb4, xb3, xb2, xb1, xb0]
    ohs = [o0h, o1h, o2h, o3h, o4h]
    obs = [ob0, ob1, ob2, ob3, ob4]
    w_offs = [int(sum(chans[:l])) for l in range(5)]
    u_offs = [int(sum(h_sizes[1:l])) for l in range(1, 5)]

    def in_copy(lvl, n, slot):
        return pltpu.make_async_copy(
            xhs[lvl].at[n], xbs[lvl].at[slot], in_sems.at[lvl, slot])

    def out_copy(lvl, n, slot):
        return pltpu.make_async_copy(
            obs[lvl].at[slot], ohs[lvl].at[n, 0], out_sems.at[lvl, slot])

    for lvl in range(5):
        in_copy(lvl, 0, 0).start()

    def body(n, _):
        slot = jax.lax.rem(n, 2)
        nslot = 1 - slot
        for lvl in range(5):
            @pl.when(n + 1 < N)
            def _prefetch(lvl=lvl, nslot=nslot):
                in_copy(lvl, n + 1, nslot).start()
            in_copy(lvl, n, slot).wait()

            @pl.when(n >= 2)
            def _drain(lvl=lvl, slot=slot):
                out_copy(lvl, n, slot).wait()

            C = chans[lvl]
            y = _wsum(xbs[lvl].at[slot], w_ref, w_offs[lvl], C)
            if lvl == 0:
                obs[0][slot] = jax.nn.sigmoid(y + b_ref[0])
            else:
                H = h_sizes[lvl]
                off = u_offs[lvl - 1]
                uh = uh_ref[:, off:off + H]
                uwt = uwt_ref[off:off + H, :]
                t = jnp.dot(uh, y, preferred_element_type=jnp.float32)
                up = jnp.dot(t, uwt, preferred_element_type=jnp.float32)
                obs[lvl][slot] = jax.nn.sigmoid(up + b_ref[lvl])
            out_copy(lvl, n, slot).start(priority=1)
        return 0

    jax.lax.fori_loop(0, N, body, 0)
    for lvl in range(5):
        out_copy(lvl, 0, 0).wait()
        out_copy(lvl, 0, 1).wait()


def kernel(x0, x1, x2, x3, x4, w0, w1, w2, w3, w4, b0, b1, b2, b3, b4):
    N = x0.shape[0]
    assert N % 2 == 0
    Ho, Wo = x4.shape[2], x4.shape[3]
    xs = [x4, x3, x2, x1, x0]
    chans = tuple(x.shape[1] for x in xs)
    h_sizes = tuple(x.shape[2] for x in xs)

    uh_all = jnp.asarray(np.concatenate(
        [_bilinear_matrix(h, Ho) for h in h_sizes[1:]], axis=1))
    uwt_all = jnp.asarray(np.concatenate(
        [_bilinear_matrix(h, Wo).T for h in h_sizes[1:]], axis=0))

    w_all = jnp.concatenate([w0, w1, w2, w3, w4])
    b_all = jnp.concatenate([b0, b1, b2, b3, b4])

    smem = pl.BlockSpec(memory_space=pltpu.MemorySpace.SMEM)
    anys = pl.BlockSpec(memory_space=pltpu.MemorySpace.HBM)
    vmem = pl.BlockSpec(memory_space=pltpu.MemorySpace.VMEM)

    out_shape = jax.ShapeDtypeStruct((N, 1, Ho, Wo), jnp.float32)
    f32 = jnp.float32

    outs = pl.pallas_call(
        functools.partial(_head_kernel, N=N, chans=chans, h_sizes=h_sizes),
        out_shape=[out_shape] * 5,
        in_specs=[smem, smem] + [anys] * 5 + [vmem, vmem],
        out_specs=[anys] * 5,
        scratch_shapes=(
            [pltpu.VMEM((2,) + x.shape[1:], f32) for x in xs]
            + [pltpu.VMEM((2, Ho, Wo), f32) for _ in range(5)]
            + [pltpu.SemaphoreType.DMA((5, 2)), pltpu.SemaphoreType.DMA((5, 2))]
        ),
    )(w_all, b_all, x4, x3, x2, x1, x0, uh_all, uwt_all)
    return list(outs)
